# Initial kernel scaffold; baseline (speedup 1.0000x reference)
import jax
import jax.numpy as jnp
from jax.experimental import pallas as pl


def kernel(input_ids, attention_mask, emb, Wih_f, Whh_f, bih_f, bhh_f, Wih_b, Whh_b, bih_b, bhh_b, gW1, gss1, gst1, gb1, gW2, gss2, gst2, gb2, ctx, linW, linb, outW, outb):
    raise NotImplementedError("write your pallas kernel here")


# speedup vs baseline: 19.4679x; 19.4679x over previous
import functools

import jax
import jax.numpy as jnp
from jax import lax
from jax.experimental import pallas as pl
from jax.experimental.pallas import tpu as pltpu
from jax.experimental.pallas import tpu_sc as plsc

D = 128
H = 64
L = 128
B = 64
NH = 8
FO = 8

_HIGHEST = lax.Precision.HIGHEST


def _sc_embedding_gather(table, idx2d):
    nchunks, chunk = idx2d.shape
    info = plsc.get_sparse_core_info()
    ncores, nsub = info.num_cores, info.num_subcores
    nw = ncores * nsub
    cpw = nchunks // nw
    mesh = plsc.VectorSubcoreMesh(core_axis_name="c", subcore_axis_name="s")

    @functools.partial(
        pl.kernel,
        mesh=mesh,
        out_type=jax.ShapeDtypeStruct((nchunks * chunk, D), jnp.float32),
        scratch_types=[
            pltpu.VMEM((cpw, chunk), jnp.int32),
            pltpu.VMEM((chunk, D), jnp.float32),
            pltpu.VMEM((chunk, D), jnp.float32),
            pltpu.SemaphoreType.DMA,
            pltpu.SemaphoreType.DMA,
        ],
    )
    def gather_kernel(table_hbm, idx_hbm, out_hbm, idx_v, rows0, rows1, sem0, sem1):
        wid = lax.axis_index("s") * ncores + lax.axis_index("c")
        pltpu.sync_copy(idx_hbm.at[pl.ds(wid * cpw, cpw)], idx_v)
        cp0 = pltpu.async_copy(table_hbm.at[idx_v.at[0]], rows0, sem0)
        cp1 = pltpu.async_copy(table_hbm.at[idx_v.at[1]], rows1, sem1)
        cp0.wait()
        pltpu.sync_copy(rows0, out_hbm.at[pl.ds(wid * cpw * chunk, chunk)])
        cp1.wait()
        pltpu.sync_copy(rows1, out_hbm.at[pl.ds((wid * cpw + 1) * chunk, chunk)])

    return gather_kernel(table, idx2d)


def _lstm_kernel(hemb_ref, wihf_ref, wihb_ref, wblk_ref, bf_ref, bb_ref,
                 hseq_ref, xwf_ref, xwb_ref):
    xwf_ref[...] = jnp.dot(hemb_ref[...], wihf_ref[...],
                           preferred_element_type=jnp.float32,
                           precision=_HIGHEST) + bf_ref[...]
    xwb_ref[...] = jnp.dot(hemb_ref[...], wihb_ref[...],
                           preferred_element_type=jnp.float32,
                           precision=_HIGHEST) + bb_ref[...]

    def sigmoid(x):
        return 1.0 / (1.0 + jnp.exp(-x))

    def body(t, carry):
        h, c = carry
        af = xwf_ref[pl.ds(t * B, B), :]
        ab = xwb_ref[pl.ds((L - 1 - t) * B, B), :]
        g = jnp.concatenate([af, ab], axis=1) + jnp.dot(
            h, wblk_ref[...], preferred_element_type=jnp.float32,
            precision=_HIGHEST)
        i_g = jnp.concatenate([g[:, 0 * H:1 * H], g[:, 4 * H:5 * H]], axis=1)
        f_g = jnp.concatenate([g[:, 1 * H:2 * H], g[:, 5 * H:6 * H]], axis=1)
        c_g = jnp.concatenate([g[:, 2 * H:3 * H], g[:, 6 * H:7 * H]], axis=1)
        o_g = jnp.concatenate([g[:, 3 * H:4 * H], g[:, 7 * H:8 * H]], axis=1)
        c_new = sigmoid(f_g) * c + sigmoid(i_g) * jnp.tanh(c_g)
        h_new = sigmoid(o_g) * jnp.tanh(c_new)
        hseq_ref[t, :, 0:H] = h_new[:, 0:H]
        hseq_ref[L - 1 - t, :, H:2 * H] = h_new[:, H:2 * H]
        return h_new, c_new

    z = jnp.zeros((B, 2 * H), jnp.float32)
    lax.fori_loop(0, L, body, (z, z))


def _gat_kernel(x_ref, gw1_ref, gt_ref, gsT_ref, gb1_ref, g2t_ref, g2sT_ref,
                ctxT_ref, mask_ref, att_ref, pooled_ref):
    x = x_ref[...]
    proj = jnp.dot(x, gw1_ref[...], preferred_element_type=jnp.float32,
                   precision=_HIGHEST)
    adst = jnp.dot(proj, gt_ref[...], preferred_element_type=jnp.float32,
                   precision=_HIGHEST)
    asrcT = lax.dot_general(gsT_ref[...], proj, (((1,), (1,)), ((), ())),
                            preferred_element_type=jnp.float32,
                            precision=_HIGHEST)
    outs = []
    for hh in range(NH):
        s = adst[:, hh:hh + 1] + asrcT[hh:hh + 1, :]
        s = jnp.where(s >= 0, s, 0.2 * s)
        mx = jnp.max(s, axis=1, keepdims=True)
        e = jnp.exp(s - mx)
        den = jnp.sum(e, axis=1, keepdims=True)
        att = e / (den + 1e-16)
        outs.append(jnp.dot(att, proj[:, FO * hh:FO * hh + FO],
                            preferred_element_type=jnp.float32,
                            precision=_HIGHEST))
    h1 = jnp.concatenate(outs, axis=1) + gb1_ref[...]
    h1 = jnp.where(h1 > 0, h1, jnp.exp(jnp.minimum(h1, 0.0)) - 1.0)

    ptc = jnp.dot(h1, g2t_ref[...], preferred_element_type=jnp.float32,
                  precision=_HIGHEST)
    psr = lax.dot_general(g2sT_ref[...], h1, (((1,), (1,)), ((), ())),
                          preferred_element_type=jnp.float32,
                          precision=_HIGHEST)
    s2 = ptc + psr
    s2 = jnp.where(s2 >= 0, s2, 0.2 * s2)
    mx2 = jnp.max(s2, axis=1, keepdims=True)
    e2 = jnp.exp(s2 - mx2)
    den2 = jnp.sum(e2, axis=1, keepdims=True)
    a2t = e2 / (den2 + 1e-16)

    att_row = jnp.dot(ctxT_ref[...], a2t, preferred_element_type=jnp.float32,
                      precision=_HIGHEST)
    m = mask_ref[0]
    att_row = jnp.where(m > 0, att_row, -jnp.inf)
    amx = jnp.max(att_row, axis=1, keepdims=True)
    ae = jnp.exp(att_row - amx)
    att_n = ae / jnp.sum(ae, axis=1, keepdims=True)
    att_ref[0] = att_n

    rr = lax.broadcasted_iota(jnp.int32, (L, L), 0)
    cc = lax.broadcasted_iota(jnp.int32, (L, L), 1)
    eye = (rr == cc).astype(jnp.float32)
    att_col = lax.dot_general(eye, att_n, (((1,), (1,)), ((), ())),
                              preferred_element_type=jnp.float32,
                              precision=_HIGHEST)
    pooled_ref[0] = jnp.max(x * att_col, axis=0, keepdims=True)


def _head_kernel(pooled_ref, linwT_ref, linb_ref, outwT_ref, outb_ref,
                 logits_ref):
    conc = jnp.dot(pooled_ref[...], linwT_ref[...],
                   preferred_element_type=jnp.float32,
                   precision=_HIGHEST) + linb_ref[...]
    conc = jnp.maximum(conc, 0.0)
    logits_ref[...] = jnp.dot(conc, outwT_ref[...],
                              preferred_element_type=jnp.float32,
                              precision=_HIGHEST) + outb_ref[...]


def kernel(input_ids, attention_mask, emb, Wih_f, Whh_f, bih_f, bhh_f,
           Wih_b, Whh_b, bih_b, bhh_b, gW1, gss1, gst1, gb1, gW2, gss2,
           gst2, gb2, ctx, linW, linb, outW, outb):
    ids_tm = input_ids.T.reshape(L * B).astype(jnp.int32)
    idx2d = ids_tm.reshape(L * B // 128, 128)

    wihf_t = Wih_f.T
    wihb_t = Wih_b.T
    bf = (bih_f + bhh_f).reshape(1, 4 * H)
    bb = (bih_b + bhh_b).reshape(1, 4 * H)
    wblk = jnp.zeros((2 * H, 8 * H), jnp.float32)
    wblk = wblk.at[0:H, 0:4 * H].set(Whh_f.T)
    wblk = wblk.at[H:2 * H, 4 * H:8 * H].set(Whh_b.T)

    eye8 = jnp.eye(NH, dtype=jnp.float32)
    gs = (gss1[0][:, :, None] * eye8[:, None, :]).reshape(NH * FO, NH)
    gt = (gst1[0][:, :, None] * eye8[:, None, :]).reshape(NH * FO, NH)
    gsT = gs.T
    g2t = gW2 * gst2.reshape(())
    g2sT = (gW2 * gss2.reshape(())).T
    ctxT = ctx.reshape(1, L)
    mask3 = attention_mask.reshape(B, 1, L).astype(jnp.float32)
    gb1r = gb1.reshape(1, NH * FO)

    h_emb = _sc_embedding_gather(emb, idx2d)

    h_seq_tm = pl.pallas_call(
        _lstm_kernel,
        out_shape=jax.ShapeDtypeStruct((L, B, 2 * H), jnp.float32),
        scratch_shapes=[
            pltpu.VMEM((L * B, 4 * H), jnp.float32),
            pltpu.VMEM((L * B, 4 * H), jnp.float32),
        ],
    )(h_emb, wihf_t, wihb_t, wblk, bf, bb)

    h2 = h_seq_tm.reshape(L, B * 2 * H)

    att3, pooled3 = pl.pallas_call(
        _gat_kernel,
        grid=(B,),
        in_specs=[
            pl.BlockSpec((L, 2 * H), lambda b: (0, b)),
            pl.BlockSpec((2 * H, NH * FO), lambda b: (0, 0)),
            pl.BlockSpec((NH * FO, NH), lambda b: (0, 0)),
            pl.BlockSpec((NH, NH * FO), lambda b: (0, 0)),
            pl.BlockSpec((1, NH * FO), lambda b: (0, 0)),
            pl.BlockSpec((NH * FO, 1), lambda b: (0, 0)),
            pl.BlockSpec((1, NH * FO), lambda b: (0, 0)),
            pl.BlockSpec((1, L), lambda b: (0, 0)),
            pl.BlockSpec((1, 1, L), lambda b: (b, 0, 0)),
        ],
        out_specs=[
            pl.BlockSpec((1, 1, L), lambda b: (b, 0, 0)),
            pl.BlockSpec((1, 1, 2 * H), lambda b: (b, 0, 0)),
        ],
        out_shape=[
            jax.ShapeDtypeStruct((B, 1, L), jnp.float32),
            jax.ShapeDtypeStruct((B, 1, 2 * H), jnp.float32),
        ],
    )(h2, gW1, gt, gsT, gb1r, g2t, g2sT, ctxT, mask3)

    att = att3.reshape(B, L)
    pooled = pooled3.reshape(B, 2 * H)

    logits = pl.pallas_call(
        _head_kernel,
        out_shape=jax.ShapeDtypeStruct((B, outW.shape[0]), jnp.float32),
    )(pooled, linW.T, linb.reshape(1, B), outW.T, outb.reshape(1, outW.shape[0]))

    return logits, att


# speedup vs baseline: 29.9259x; 1.5372x over previous
import functools

import jax
import jax.numpy as jnp
from jax import lax
from jax.experimental import pallas as pl
from jax.experimental.pallas import tpu as pltpu
from jax.experimental.pallas import tpu_sc as plsc

D = 128
H = 64
L = 128
B = 64
NH = 8
FO = 8


def _sc_embedding_gather(table, idx2d):
    nchunks, chunk = idx2d.shape
    info = plsc.get_sparse_core_info()
    ncores, nsub = info.num_cores, info.num_subcores
    nw = ncores * nsub
    cpw = nchunks // nw
    mesh = plsc.VectorSubcoreMesh(core_axis_name="c", subcore_axis_name="s")

    @functools.partial(
        pl.kernel,
        mesh=mesh,
        out_type=jax.ShapeDtypeStruct((nchunks * chunk, D), jnp.float32),
        scratch_types=[
            pltpu.VMEM((cpw, chunk), jnp.int32),
            pltpu.VMEM((chunk, D), jnp.float32),
            pltpu.VMEM((chunk, D), jnp.float32),
            pltpu.SemaphoreType.DMA,
            pltpu.SemaphoreType.DMA,
        ],
    )
    def gather_kernel(table_hbm, idx_hbm, out_hbm, idx_v, rows0, rows1, sem0, sem1):
        wid = lax.axis_index("s") * ncores + lax.axis_index("c")
        pltpu.sync_copy(idx_hbm.at[pl.ds(wid * cpw, cpw)], idx_v)
        cp0 = pltpu.async_copy(table_hbm.at[idx_v.at[0]], rows0, sem0)
        cp1 = pltpu.async_copy(table_hbm.at[idx_v.at[1]], rows1, sem1)
        cp0.wait()
        pltpu.sync_copy(rows0, out_hbm.at[pl.ds(wid * cpw * chunk, chunk)])
        cp1.wait()
        pltpu.sync_copy(rows1, out_hbm.at[pl.ds((wid * cpw + 1) * chunk, chunk)])

    return gather_kernel(table, idx2d)


def _lstm_kernel(hemb_ref, wihf_ref, wihb_ref, wblk_ref, bf_ref, bb_ref,
                 hseq_ref, xwf_ref, xwb_ref):
    xwf_ref[...] = jnp.dot(hemb_ref[...], wihf_ref[...],
                           preferred_element_type=jnp.float32) + bf_ref[...]
    xwb_ref[...] = jnp.dot(hemb_ref[...], wihb_ref[...],
                           preferred_element_type=jnp.float32) + bb_ref[...]

    def sigmoid(x):
        return 1.0 / (1.0 + jnp.exp(-x))

    def body(t, carry):
        h, c = carry
        af = xwf_ref[pl.ds(t * B, B), :]
        ab = xwb_ref[pl.ds((L - 1 - t) * B, B), :]
        g = jnp.concatenate([af, ab], axis=1) + jnp.dot(
            h, wblk_ref[...], preferred_element_type=jnp.float32)
        i_g = jnp.concatenate([g[:, 0 * H:1 * H], g[:, 4 * H:5 * H]], axis=1)
        f_g = jnp.concatenate([g[:, 1 * H:2 * H], g[:, 5 * H:6 * H]], axis=1)
        c_g = jnp.concatenate([g[:, 2 * H:3 * H], g[:, 6 * H:7 * H]], axis=1)
        o_g = jnp.concatenate([g[:, 3 * H:4 * H], g[:, 7 * H:8 * H]], axis=1)
        c_new = sigmoid(f_g) * c + sigmoid(i_g) * jnp.tanh(c_g)
        h_new = sigmoid(o_g) * jnp.tanh(c_new)
        hseq_ref[t, :, 0:H] = h_new[:, 0:H]
        hseq_ref[L - 1 - t, :, H:2 * H] = h_new[:, H:2 * H]
        return h_new, c_new

    z = jnp.zeros((B, 2 * H), jnp.float32)
    lax.fori_loop(0, L, body, (z, z))


SPG = 4


def _gat_kernel(x_ref, gw1_ref, gt_ref, gsT_ref, gb1_ref, g2t_ref, g2sT_ref,
                ctxT_ref, mask_ref, att_ref, pooled_ref):
    rr = lax.broadcasted_iota(jnp.int32, (L, L), 0)
    cc = lax.broadcasted_iota(jnp.int32, (L, L), 1)
    eye = (rr == cc).astype(jnp.float32)
    for sidx in range(SPG):
        x = x_ref[:, sidx * 2 * H:(sidx + 1) * 2 * H]
        proj = jnp.dot(x, gw1_ref[...], preferred_element_type=jnp.float32)
        adst = jnp.dot(proj, gt_ref[...], preferred_element_type=jnp.float32)
        asrcT = lax.dot_general(gsT_ref[...], proj, (((1,), (1,)), ((), ())),
                                preferred_element_type=jnp.float32)
        outs = []
        for hh in range(NH):
            s = adst[:, hh:hh + 1] + asrcT[hh:hh + 1, :]
            s = jnp.where(s >= 0, s, 0.2 * s)
            mx = jnp.max(s, axis=1, keepdims=True)
            e = jnp.exp(s - mx)
            den = jnp.sum(e, axis=1, keepdims=True)
            att = e / (den + 1e-16)
            outs.append(jnp.dot(att, proj[:, FO * hh:FO * hh + FO],
                                preferred_element_type=jnp.float32))
        h1 = jnp.concatenate(outs, axis=1) + gb1_ref[...]
        h1 = jnp.where(h1 > 0, h1, jnp.exp(jnp.minimum(h1, 0.0)) - 1.0)

        ptc = jnp.dot(h1, g2t_ref[...], preferred_element_type=jnp.float32)
        psr = lax.dot_general(g2sT_ref[...], h1, (((1,), (1,)), ((), ())),
                              preferred_element_type=jnp.float32)
        s2 = ptc + psr
        s2 = jnp.where(s2 >= 0, s2, 0.2 * s2)
        mx2 = jnp.max(s2, axis=1, keepdims=True)
        e2 = jnp.exp(s2 - mx2)
        den2 = jnp.sum(e2, axis=1, keepdims=True)
        a2t = e2 / (den2 + 1e-16)

        att_row = jnp.dot(ctxT_ref[...], a2t, preferred_element_type=jnp.float32)
        m = mask_ref[sidx]
        att_row = jnp.where(m > 0, att_row, -jnp.inf)
        amx = jnp.max(att_row, axis=1, keepdims=True)
        ae = jnp.exp(att_row - amx)
        att_n = ae / jnp.sum(ae, axis=1, keepdims=True)
        att_ref[sidx] = att_n

        att_col = lax.dot_general(eye, att_n, (((1,), (1,)), ((), ())),
                                  preferred_element_type=jnp.float32)
        pooled_ref[sidx] = jnp.max(x * att_col, axis=0, keepdims=True)


def _head_kernel(pooled_ref, linwT_ref, linb_ref, outwT_ref, outb_ref,
                 logits_ref):
    conc = jnp.dot(pooled_ref[...], linwT_ref[...],
                   preferred_element_type=jnp.float32) + linb_ref[...]
    conc = jnp.maximum(conc, 0.0)
    logits_ref[...] = jnp.dot(conc, outwT_ref[...],
                              preferred_element_type=jnp.float32) + outb_ref[...]


def kernel(input_ids, attention_mask, emb, Wih_f, Whh_f, bih_f, bhh_f,
           Wih_b, Whh_b, bih_b, bhh_b, gW1, gss1, gst1, gb1, gW2, gss2,
           gst2, gb2, ctx, linW, linb, outW, outb):
    ids_tm = input_ids.T.reshape(L * B).astype(jnp.int32)
    idx2d = ids_tm.reshape(L * B // 128, 128)

    wihf_t = Wih_f.T
    wihb_t = Wih_b.T
    bf = (bih_f + bhh_f).reshape(1, 4 * H)
    bb = (bih_b + bhh_b).reshape(1, 4 * H)
    wblk = jnp.zeros((2 * H, 8 * H), jnp.float32)
    wblk = wblk.at[0:H, 0:4 * H].set(Whh_f.T)
    wblk = wblk.at[H:2 * H, 4 * H:8 * H].set(Whh_b.T)

    eye8 = jnp.eye(NH, dtype=jnp.float32)
    gs = (gss1[0][:, :, None] * eye8[:, None, :]).reshape(NH * FO, NH)
    gt = (gst1[0][:, :, None] * eye8[:, None, :]).reshape(NH * FO, NH)
    gsT = gs.T
    g2t = gW2 * gst2.reshape(())
    g2sT = (gW2 * gss2.reshape(())).T
    ctxT = ctx.reshape(1, L)
    mask3 = attention_mask.reshape(B, 1, L).astype(jnp.float32)
    gb1r = gb1.reshape(1, NH * FO)

    h_emb = _sc_embedding_gather(emb, idx2d)

    h_seq_tm = pl.pallas_call(
        _lstm_kernel,
        out_shape=jax.ShapeDtypeStruct((L, B, 2 * H), jnp.float32),
        scratch_shapes=[
            pltpu.VMEM((L * B, 4 * H), jnp.float32),
            pltpu.VMEM((L * B, 4 * H), jnp.float32),
        ],
    )(h_emb, wihf_t, wihb_t, wblk, bf, bb)

    h2 = h_seq_tm.reshape(L, B * 2 * H)

    att3, pooled3 = pl.pallas_call(
        _gat_kernel,
        grid=(B // SPG,),
        in_specs=[
            pl.BlockSpec((L, SPG * 2 * H), lambda b: (0, b)),
            pl.BlockSpec((2 * H, NH * FO), lambda b: (0, 0)),
            pl.BlockSpec((NH * FO, NH), lambda b: (0, 0)),
            pl.BlockSpec((NH, NH * FO), lambda b: (0, 0)),
            pl.BlockSpec((1, NH * FO), lambda b: (0, 0)),
            pl.BlockSpec((NH * FO, 1), lambda b: (0, 0)),
            pl.BlockSpec((1, NH * FO), lambda b: (0, 0)),
            pl.BlockSpec((1, L), lambda b: (0, 0)),
            pl.BlockSpec((SPG, 1, L), lambda b: (b, 0, 0)),
        ],
        out_specs=[
            pl.BlockSpec((SPG, 1, L), lambda b: (b, 0, 0)),
            pl.BlockSpec((SPG, 1, 2 * H), lambda b: (b, 0, 0)),
        ],
        out_shape=[
            jax.ShapeDtypeStruct((B, 1, L), jnp.float32),
            jax.ShapeDtypeStruct((B, 1, 2 * H), jnp.float32),
        ],
    )(h2, gW1, gt, gsT, gb1r, g2t, g2sT, ctxT, mask3)

    att = att3.reshape(B, L)
    pooled = pooled3.reshape(B, 2 * H)

    logits = pl.pallas_call(
        _head_kernel,
        out_shape=jax.ShapeDtypeStruct((B, outW.shape[0]), jnp.float32),
    )(pooled, linW.T, linb.reshape(1, B), outW.T, outb.reshape(1, outW.shape[0]))

    return logits, att
